# Initial kernel scaffold; baseline (speedup 1.0000x reference)
import jax
import jax.numpy as jnp
from jax.experimental import pallas as pl


def kernel(x, emb, W1, b1, W2, b2):
    raise NotImplementedError("write your pallas kernel here")


# speedup vs baseline: 16.7064x; 16.7064x over previous
import functools

import jax
import jax.numpy as jnp
from jax import lax
from jax.experimental import pallas as pl
from jax.experimental.pallas import tpu as pltpu
from jax.experimental.pallas import tpu_sc as plsc

B = 16384
L = 200
EMBED = 50
DP = 64
HIDDEN = 256
OUT = 2

NC = 2
NS = 16
NW = NC * NS
S_PER_W = B // NW
CS = 4
CHUNKS = S_PER_W // CS
ROWS_PER_CHUNK = CS * L
IDX_W = 100
GATHERS = ROWS_PER_CHUNK // IDX_W


def _pool_body(xr_hbm, emb_hbm, out_hbm, idx_v, rows_v, acc_v, sem):
    wid = lax.axis_index("s") * NC + lax.axis_index("c")

    def chunk_body(c, carry):
        xbase = wid * (S_PER_W * 2) + c * (CS * 2)
        pltpu.sync_copy(xr_hbm.at[pl.ds(xbase, CS * 2)], idx_v)
        copies = [
            pltpu.async_copy(
                emb_hbm.at[idx_v.at[j]],
                rows_v.at[pl.ds(j * IDX_W, IDX_W)],
                sem,
            )
            for j in range(GATHERS)
        ]
        for cp in copies:
            cp.wait()

        for s in range(CS):
            base = s * L

            def accum(l, accs):
                accs = list(accs)
                for u in range(8):
                    row = base + l * 8 + u
                    for w in range(4):
                        accs[w] = accs[w] + rows_v[row, pl.ds(w * 16, 16)]
                return tuple(accs)

            zeros = tuple(jnp.zeros((16,), jnp.float32) for _ in range(4))
            accs = lax.fori_loop(0, L // 8, accum, zeros)
            for w in range(4):
                acc_v[s, pl.ds(w * 16, 16)] = accs[w]

        pltpu.sync_copy(acc_v, out_hbm.at[pl.ds(wid * S_PER_W + c * CS, CS)])
        return carry

    lax.fori_loop(0, CHUNKS, chunk_body, 0)


@functools.partial(
    pl.kernel,
    out_type=jax.ShapeDtypeStruct((B, DP), jnp.float32),
    mesh=plsc.VectorSubcoreMesh(core_axis_name="c", subcore_axis_name="s"),
    scratch_types=[
        pltpu.VMEM((CS * 2, IDX_W), jnp.int32),
        pltpu.VMEM((ROWS_PER_CHUNK, DP), jnp.float32),
        pltpu.VMEM((CS, DP), jnp.float32),
        pltpu.SemaphoreType.DMA,
    ],
    compiler_params=pltpu.CompilerParams(use_tc_tiling_on_sc=False),
)
def _pool(xr_hbm, emb_hbm, out_hbm, idx_v, rows_v, acc_v, sem):
    _pool_body(xr_hbm, emb_hbm, out_hbm, idx_v, rows_v, acc_v, sem)


def _mlp_body(p_ref, w1_ref, b1_ref, w2_ref, b2_ref, o_ref):
    h = jnp.dot(p_ref[...], w1_ref[...], preferred_element_type=jnp.float32)
    h = jnp.maximum(h + b1_ref[...], 0.0)
    logits = jnp.dot(h, w2_ref[...], preferred_element_type=jnp.float32)
    logits = logits + b2_ref[...]
    m = jnp.max(logits, axis=1, keepdims=True)
    ls = jnp.log(jnp.sum(jnp.exp(logits - m), axis=1, keepdims=True)) + m
    o_ref[...] = logits - ls


_MLP_BS = 2048


def _mlp(pooled, w1, b1, w2, b2):
    return pl.pallas_call(
        _mlp_body,
        grid=(B // _MLP_BS,),
        in_specs=[
            pl.BlockSpec((_MLP_BS, DP), lambda i: (i, 0)),
            pl.BlockSpec((DP, HIDDEN), lambda i: (0, 0)),
            pl.BlockSpec((1, HIDDEN), lambda i: (0, 0)),
            pl.BlockSpec((HIDDEN, OUT), lambda i: (0, 0)),
            pl.BlockSpec((1, OUT), lambda i: (0, 0)),
        ],
        out_specs=pl.BlockSpec((_MLP_BS, OUT), lambda i: (i, 0)),
        out_shape=jax.ShapeDtypeStruct((B, OUT), jnp.float32),
    )(pooled, w1, b1, w2, b2)


def kernel(x, emb, W1, b1, W2, b2):
    xr = x.reshape(B * 2, L // 2).astype(jnp.int32)
    embp = jnp.pad(emb, ((0, 0), (0, DP - EMBED)))
    pooled = _pool(xr, embp)
    w1 = jnp.pad(W1.T, ((0, DP - EMBED), (0, 0))) * (1.0 / L)
    out = _mlp(pooled, w1, b1.reshape(1, HIDDEN), W2.T, b2.reshape(1, OUT))
    return out


# speedup vs baseline: 25.8118x; 1.5450x over previous
import functools

import jax
import jax.numpy as jnp
from jax import lax
from jax.experimental import pallas as pl
from jax.experimental.pallas import tpu as pltpu
from jax.experimental.pallas import tpu_sc as plsc

B = 16384
L = 200
EMBED = 50
DP = 64
HIDDEN = 256
OUT = 2

NC = 2
NS = 16
NW = NC * NS
S_PER_W = B // NW
CS = 4
CHUNKS = S_PER_W // CS
ROWS_PER_CHUNK = CS * L
IDX_W = 100
GATHERS = ROWS_PER_CHUNK // IDX_W


def _pool_body(xr_hbm, emb_hbm, out_hbm, idx_v, rows_v, acc_v, sem0, sem1):
    wid = lax.axis_index("s") * NC + lax.axis_index("c")
    xw = wid * (S_PER_W * 2)
    ow = wid * S_PER_W
    sems = (sem0, sem1)

    def load_and_fire(c, buf):
        pltpu.sync_copy(
            xr_hbm.at[pl.ds(xw + c * (CS * 2), CS * 2)],
            idx_v.at[pl.ds(buf * GATHERS, GATHERS)],
        )
        for j in range(GATHERS):
            pltpu.async_copy(
                emb_hbm.at[idx_v.at[buf * GATHERS + j]],
                rows_v.at[pl.ds(buf * ROWS_PER_CHUNK + j * IDX_W, IDX_W)],
                sems[buf],
            )

    def wait_buf(buf):
        pltpu.make_async_copy(
            emb_hbm.at[pl.ds(0, ROWS_PER_CHUNK)],
            rows_v.at[pl.ds(buf * ROWS_PER_CHUNK, ROWS_PER_CHUNK)],
            sems[buf],
        ).wait()

    def accum_store(c, buf):
        rbase = buf * ROWS_PER_CHUNK
        for s in range(CS):
            base = rbase + s * L

            def accum(l, accs):
                accs = list(accs)
                for u in range(8):
                    row = base + l * 8 + u
                    for w in range(4):
                        accs[w] = accs[w] + rows_v[row, pl.ds(w * 16, 16)]
                return tuple(accs)

            zeros = tuple(jnp.zeros((16,), jnp.float32) for _ in range(4))
            accs = lax.fori_loop(0, L // 8, accum, zeros)
            for w in range(4):
                acc_v[s, pl.ds(w * 16, 16)] = accs[w]
        pltpu.sync_copy(acc_v, out_hbm.at[pl.ds(ow + c * CS, CS)])

    load_and_fire(0, 0)

    def pair_body(cp, carry):
        c0 = cp * 2
        load_and_fire(c0 + 1, 1)
        wait_buf(0)
        accum_store(c0, 0)

        @pl.when(cp < CHUNKS // 2 - 1)
        def _():
            load_and_fire(c0 + 2, 0)

        wait_buf(1)
        accum_store(c0 + 1, 1)
        return carry

    lax.fori_loop(0, CHUNKS // 2, pair_body, 0)


@functools.partial(
    pl.kernel,
    out_type=jax.ShapeDtypeStruct((B, DP), jnp.float32),
    mesh=plsc.VectorSubcoreMesh(core_axis_name="c", subcore_axis_name="s"),
    scratch_types=[
        pltpu.VMEM((2 * GATHERS, IDX_W), jnp.int32),
        pltpu.VMEM((2 * ROWS_PER_CHUNK, DP), jnp.float32),
        pltpu.VMEM((CS, DP), jnp.float32),
        pltpu.SemaphoreType.DMA,
        pltpu.SemaphoreType.DMA,
    ],
    compiler_params=pltpu.CompilerParams(use_tc_tiling_on_sc=False),
)
def _pool(xr_hbm, emb_hbm, out_hbm, idx_v, rows_v, acc_v, sem0, sem1):
    _pool_body(xr_hbm, emb_hbm, out_hbm, idx_v, rows_v, acc_v, sem0, sem1)


def _mlp_body(p_ref, w1_ref, b1_ref, w2_ref, b2_ref, o_ref):
    h = jnp.dot(p_ref[...], w1_ref[...], preferred_element_type=jnp.float32)
    h = jnp.maximum(h + b1_ref[...], 0.0)
    logits = jnp.dot(h, w2_ref[...], preferred_element_type=jnp.float32)
    logits = logits + b2_ref[...]
    m = jnp.max(logits, axis=1, keepdims=True)
    ls = jnp.log(jnp.sum(jnp.exp(logits - m), axis=1, keepdims=True)) + m
    o_ref[...] = logits - ls


_MLP_BS = 2048


def _mlp(pooled, w1, b1, w2, b2):
    return pl.pallas_call(
        _mlp_body,
        grid=(B // _MLP_BS,),
        in_specs=[
            pl.BlockSpec((_MLP_BS, DP), lambda i: (i, 0)),
            pl.BlockSpec((DP, HIDDEN), lambda i: (0, 0)),
            pl.BlockSpec((1, HIDDEN), lambda i: (0, 0)),
            pl.BlockSpec((HIDDEN, OUT), lambda i: (0, 0)),
            pl.BlockSpec((1, OUT), lambda i: (0, 0)),
        ],
        out_specs=pl.BlockSpec((_MLP_BS, OUT), lambda i: (i, 0)),
        out_shape=jax.ShapeDtypeStruct((B, OUT), jnp.float32),
    )(pooled, w1, b1, w2, b2)


def kernel(x, emb, W1, b1, W2, b2):
    xr = x.reshape(B * 2, L // 2).astype(jnp.int32)
    embp = jnp.pad(emb, ((0, 0), (0, DP - EMBED)))
    pooled = _pool(xr, embp)
    w1 = jnp.pad(W1.T, ((0, DP - EMBED), (0, 0))) * (1.0 / L)
    out = _mlp(pooled, w1, b1.reshape(1, HIDDEN), W2.T, b2.reshape(1, OUT))
    return out


# speedup vs baseline: 26.5356x; 1.0280x over previous
import functools

import jax
import jax.numpy as jnp
from jax import lax
from jax.experimental import pallas as pl
from jax.experimental.pallas import tpu as pltpu
from jax.experimental.pallas import tpu_sc as plsc

B = 16384
L = 200
EMBED = 50
DP = 64
HIDDEN = 256
OUT = 2

NC = 2
NS = 16
NW = NC * NS
S_PER_W = B // NW
CS = 8
CHUNKS = S_PER_W // CS
R = 4
W = L // R
GATHERS = CS * R
ROWS_PER_CHUNK = CS * W
UNROLL = 5


def _pool_body(xr_hbm, emb_hbm, out_hbm, idx_v, rows_v, acc_v, sem0, sem1):
    wid = lax.axis_index("s") * NC + lax.axis_index("c")
    xw = wid * (S_PER_W * R)
    ow = wid * S_PER_W
    sems = (sem0, sem1)
    zero = jnp.zeros((16,), jnp.float32)

    def zero_all(i, carry):
        for w in range(4):
            rows_v[i, pl.ds(w * 16, 16)] = zero
        return carry

    lax.fori_loop(0, 2 * ROWS_PER_CHUNK, zero_all, 0)

    def load_and_fire(c, buf):
        pltpu.sync_copy(
            xr_hbm.at[pl.ds(xw + c * (CS * R), CS * R)],
            idx_v.at[pl.ds(buf * GATHERS, GATHERS)],
        )
        for j in range(GATHERS):
            pltpu.async_copy(
                emb_hbm.at[idx_v.at[buf * GATHERS + j]],
                rows_v.at[pl.ds(buf * ROWS_PER_CHUNK + (j // R) * W, W)],
                sems[buf],
                add=True,
            )

    def wait_buf(buf):
        for _ in range(R):
            pltpu.make_async_copy(
                emb_hbm.at[pl.ds(0, ROWS_PER_CHUNK)],
                rows_v.at[pl.ds(buf * ROWS_PER_CHUNK, ROWS_PER_CHUNK)],
                sems[buf],
            ).wait()

    def accum_store(c, buf):
        rbase = buf * ROWS_PER_CHUNK
        for s in range(CS):
            base = rbase + s * W

            def accum(l, accs):
                accs = list(accs)
                for u in range(UNROLL):
                    row = base + l * UNROLL + u
                    for w in range(4):
                        accs[w] = accs[w] + rows_v[row, pl.ds(w * 16, 16)]
                        rows_v[row, pl.ds(w * 16, 16)] = zero
                return tuple(accs)

            accs = lax.fori_loop(0, W // UNROLL, accum, (zero,) * 4)
            for w in range(4):
                acc_v[s, pl.ds(w * 16, 16)] = accs[w]
        pltpu.sync_copy(acc_v, out_hbm.at[pl.ds(ow + c * CS, CS)])

    load_and_fire(0, 0)

    def pair_body(cp, carry):
        c0 = cp * 2
        load_and_fire(c0 + 1, 1)
        wait_buf(0)
        accum_store(c0, 0)

        @pl.when(cp < CHUNKS // 2 - 1)
        def _():
            load_and_fire(c0 + 2, 0)

        wait_buf(1)
        accum_store(c0 + 1, 1)
        return carry

    lax.fori_loop(0, CHUNKS // 2, pair_body, 0)


@functools.partial(
    pl.kernel,
    out_type=jax.ShapeDtypeStruct((B, DP), jnp.float32),
    mesh=plsc.VectorSubcoreMesh(core_axis_name="c", subcore_axis_name="s"),
    scratch_types=[
        pltpu.VMEM((2 * GATHERS, W), jnp.int32),
        pltpu.VMEM((2 * ROWS_PER_CHUNK, DP), jnp.float32),
        pltpu.VMEM((CS, DP), jnp.float32),
        pltpu.SemaphoreType.DMA,
        pltpu.SemaphoreType.DMA,
    ],
    compiler_params=pltpu.CompilerParams(use_tc_tiling_on_sc=False),
)
def _pool(xr_hbm, emb_hbm, out_hbm, idx_v, rows_v, acc_v, sem0, sem1):
    _pool_body(xr_hbm, emb_hbm, out_hbm, idx_v, rows_v, acc_v, sem0, sem1)


def _mlp_body(p_ref, w1_ref, b1_ref, w2_ref, b2_ref, o_ref):
    h = jnp.dot(p_ref[...], w1_ref[...], preferred_element_type=jnp.float32)
    h = jnp.maximum(h + b1_ref[...], 0.0)
    logits = jnp.dot(h, w2_ref[...], preferred_element_type=jnp.float32)
    logits = logits + b2_ref[...]
    m = jnp.max(logits, axis=1, keepdims=True)
    ls = jnp.log(jnp.sum(jnp.exp(logits - m), axis=1, keepdims=True)) + m
    o_ref[...] = logits - ls


_MLP_BS = 2048


def _mlp(pooled, w1, b1, w2, b2):
    return pl.pallas_call(
        _mlp_body,
        grid=(B // _MLP_BS,),
        in_specs=[
            pl.BlockSpec((_MLP_BS, DP), lambda i: (i, 0)),
            pl.BlockSpec((DP, HIDDEN), lambda i: (0, 0)),
            pl.BlockSpec((1, HIDDEN), lambda i: (0, 0)),
            pl.BlockSpec((HIDDEN, OUT), lambda i: (0, 0)),
            pl.BlockSpec((1, OUT), lambda i: (0, 0)),
        ],
        out_specs=pl.BlockSpec((_MLP_BS, OUT), lambda i: (i, 0)),
        out_shape=jax.ShapeDtypeStruct((B, OUT), jnp.float32),
    )(pooled, w1, b1, w2, b2)


def kernel(x, emb, W1, b1, W2, b2):
    xr = x.reshape(B * R, W).astype(jnp.int32)
    embp = jnp.pad(emb, ((0, 0), (0, DP - EMBED)))
    pooled = _pool(xr, embp)
    w1 = jnp.pad(W1.T, ((0, DP - EMBED), (0, 0))) * (1.0 / L)
    out = _mlp(pooled, w1, b1.reshape(1, HIDDEN), W2.T, b2.reshape(1, OUT))
    return out
